# baseline (speedup 1.0000x reference)
import functools

import jax
import jax.numpy as jnp
from jax import lax
from jax.experimental import pallas as pl
from jax.experimental.pallas import tpu as pltpu
from jax.experimental.pallas import tpu_sc as plsc

N = 10000
E = 320000
D = 128

NC = 2
NS = 16
NW = NC * NS
EPW = E // NW
K = 80
NCHUNK = EPW // K
NB = 4
NSB = 2
NQ = 8
NOUT = -(-NCHUNK // NQ)
N_PAD = 10240
RPT = N_PAD // NS
LANES = 16


def _sc_segment_sum_one(items_emb, zeros, srcv, dstv, wv):
    mesh = plsc.VectorSubcoreMesh(core_axis_name="c", subcore_axis_name="s")

    @functools.partial(
        pl.kernel,
        mesh=mesh,
        out_type=jax.ShapeDtypeStruct((NC, N_PAD, D), jnp.float32),
        scratch_types=(
            [pltpu.VMEM((NQ, K), jnp.int32),
             pltpu.VMEM((NQ, K), jnp.int32),
             pltpu.VMEM((NQ, K), jnp.float32)]
            + [pltpu.VMEM((K, D), jnp.float32) for _ in range(NB)]
            + [pltpu.VMEM_SHARED((N_PAD, D), jnp.float32)]
            + [pltpu.SemaphoreType.DMA for _ in range(NQ + 2 * NB + 1)]
        ),
    )
    def seg(emb_hbm, z_hbm, src_hbm, dst_hbm, wr_hbm,
            out_hbm, sring, dring, wring, *rest):
        rows = list(rest[:NB])
        acc = rest[NB]
        base = NB + 1
        isem = list(rest[base:base + NQ])
        gsem = list(rest[base + NQ:base + NQ + NB])
        ssem = list(rest[base + NQ + NB:base + NQ + 2 * NB])
        zsem = rest[base + NQ + 2 * NB]

        cid = lax.axis_index("c")
        sid = lax.axis_index("s")
        wid = sid * NC + cid
        ebase = wid * EPW
        row0 = sid * RPT

        if True:
            def load_idx(c, q):
                e0 = ebase + c * K
                pltpu.async_copy(src_hbm.at[pl.ds(e0, K)], sring.at[q],
                                 isem[q])
                pltpu.async_copy(dst_hbm.at[pl.ds(e0, K)], dring.at[q],
                                 isem[q])
                pltpu.async_copy(wr_hbm.at[pl.ds(e0, K)], wring.at[q],
                                 isem[q])

            def wait_idx(c, q):
                e0 = ebase + c * K
                pltpu.make_async_copy(src_hbm.at[pl.ds(e0, K)],
                                      sring.at[q], isem[q]).wait()
                pltpu.make_async_copy(dst_hbm.at[pl.ds(e0, K)],
                                      dring.at[q], isem[q]).wait()
                pltpu.make_async_copy(wr_hbm.at[pl.ds(e0, K)],
                                      wring.at[q], isem[q]).wait()

            pltpu.async_copy(z_hbm, acc.at[pl.ds(row0, RPT)], zsem)
            for p in range(NQ - 1):
                load_idx(p, p)
            for p in range(NB - 1):
                wait_idx(p, p)
                pltpu.async_copy(emb_hbm.at[sring.at[p]], rows[p], gsem[p])
            pltpu.make_async_copy(z_hbm, acc.at[pl.ds(row0, RPT)],
                                  zsem).wait()
            plsc.subcore_barrier()

            def outer(i, _):
                c0v = i * NQ
                for b in range(NQ):
                    c = c0v + b
                    rb = b % NB
                    rp = (b + NB - 1) % NB

                    @pl.when(c < NCHUNK)
                    def _(c=c, b=b, rb=rb, rp=rp):
                        buf = rows[rb]
                        pltpu.make_async_copy(
                            emb_hbm.at[sring.at[b]], buf, gsem[rb]).wait()

                        def scale(g, _):
                            wv = wring[b, pl.ds(g * LANES, LANES)]
                            for t in range(LANES):
                                e = g * LANES + t
                                ws = wv[t]
                                for cc in range(D // LANES):
                                    sl = pl.ds(cc * LANES, LANES)
                                    buf[e, sl] = buf[e, sl] * ws
                            return 0
                        lax.fori_loop(0, K // LANES, scale, 0)

                        @pl.when(c >= 1)
                        def _():
                            pltpu.make_async_copy(
                                rows[rp], acc.at[dring.at[(b - 1) % NQ]],
                                ssem[rp]).wait()
                        pltpu.async_copy(buf, acc.at[dring.at[b]],
                                         ssem[rb], add=True)

                        @pl.when(c + NB - 1 < NCHUNK)
                        def _():
                            q2 = (b + NB - 1) % NQ
                            wait_idx(c + NB - 1, q2)
                            pltpu.async_copy(emb_hbm.at[sring.at[q2]],
                                             rows[rp], gsem[rp])

                        @pl.when(c + NQ - 1 < NCHUNK)
                        def _():
                            q3 = (b + NQ - 1) % NQ
                            load_idx(c + NQ - 1, q3)
                return 0
            lax.fori_loop(0, NOUT, outer, 0)
            pltpu.make_async_copy(
                rows[(NCHUNK - 1) % NB],
                acc.at[dring.at[(NCHUNK - 1) % NQ]],
                ssem[(NCHUNK - 1) % NB]).wait()
            plsc.subcore_barrier()

            pltpu.sync_copy(acc.at[pl.ds(row0, RPT)],
                            out_hbm.at[cid, pl.ds(row0, RPT)])

    return seg(items_emb, zeros, srcv, dstv, wv)


ROWS_BLK = 1000
SCALE = 1.0 / (D ** 0.5)


def _rel_body(x_ref, part_ref, I_ref, nb_ref, s_ref):
    x = x_ref[...]
    nb = part_ref[0] + part_ref[1]
    h = jnp.dot(x * nb, I_ref[...],
                preferred_element_type=jnp.float32,
                precision=lax.Precision.HIGHEST)
    h = jnp.where(h > 0, h, 0.2 * h)
    nb_ref[...] = nb
    s_ref[...] = jnp.broadcast_to(
        jnp.sum(h, axis=1, keepdims=True) * SCALE, h.shape)


def _tc_rel(items_emb, part, I):
    grid = (N // ROWS_BLK,)
    return pl.pallas_call(
        _rel_body,
        grid=grid,
        in_specs=[
            pl.BlockSpec((ROWS_BLK, D), lambda i: (i, 0)),
            pl.BlockSpec((NC, ROWS_BLK, D), lambda i: (0, i, 0)),
            pl.BlockSpec((D, D), lambda i: (0, 0)),
        ],
        out_specs=[pl.BlockSpec((ROWS_BLK, D), lambda i: (i, 0)),
                   pl.BlockSpec((ROWS_BLK, D), lambda i: (i, 0))],
        out_shape=[jax.ShapeDtypeStruct((N, D), jnp.float32),
                   jax.ShapeDtypeStruct((N, D), jnp.float32)],
    )(items_emb, part, I)


def _final_body(x_ref, n0_ref, s0_ref, n1_ref, s1_ref, n2_ref, s2_ref,
                cw_ref, cb_ref, out_ref):
    x = x_ref[...]
    nb = [n0_ref[...], n1_ref[...], n2_ref[...]]
    s = [s0_ref[:, 0:1], s1_ref[:, 0:1], s2_ref[:, 0:1]]
    m = jnp.maximum(jnp.maximum(s[0], s[1]), s[2])
    e = [jnp.exp(sr - m) for sr in s]
    denom = e[0] + e[1] + e[2]
    neighbor = (nb[0] * e[0] + nb[1] * e[1] + nb[2] * e[2]) / denom
    out_ref[...] = x * cw_ref[0] + neighbor * cw_ref[1] + cb_ref[0]


def _tc_final(items_emb, rels, conv_w, conv_b):
    grid = (N // ROWS_BLK,)
    blk = pl.BlockSpec((ROWS_BLK, D), lambda i: (i, 0))
    flat = []
    for nb_r, s_r in rels:
        flat += [nb_r, s_r]
    return pl.pallas_call(
        _final_body,
        grid=grid,
        in_specs=[blk] * 7 + [
            pl.BlockSpec(memory_space=pltpu.SMEM),
            pl.BlockSpec(memory_space=pltpu.SMEM),
        ],
        out_specs=blk,
        out_shape=jax.ShapeDtypeStruct((N, D), jnp.float32),
    )(items_emb, *flat, conv_w, conv_b)


def kernel(b, items_emb, p2p_in_idx, p2p_in_w, p2p_out_idx, p2p_out_w,
           e2p_in_idx, e2p_in_w, I_p2p_in, I_p2p_out, I_e2p_in, conv_w,
           conv_b):
    del b

    zeros = jnp.zeros((RPT, D), jnp.float32)
    rels = []
    for idx, w, I in ((p2p_in_idx, p2p_in_w, I_p2p_in),
                      (p2p_out_idx, p2p_out_w, I_p2p_out),
                      (e2p_in_idx, e2p_in_w, I_e2p_in)):
        part = _sc_segment_sum_one(items_emb, zeros, idx[1], idx[0], w)
        rels.append(_tc_rel(items_emb, part, I))
    return _tc_final(items_emb, rels, conv_w, jnp.reshape(conv_b, (1,)))


# speedup vs baseline: 1.0218x; 1.0218x over previous
import functools

import jax
import jax.numpy as jnp
from jax import lax
from jax.experimental import pallas as pl
from jax.experimental.pallas import tpu as pltpu
from jax.experimental.pallas import tpu_sc as plsc

N = 10000
E = 320000
D = 128

NC = 2
NS = 16
NW = NC * NS
EPW = E // NW
K = 80
NCHUNK = EPW // K
NB = 4
NQ = 8
NOUT = -(-NCHUNK // NQ)
N_PAD = 10240
RPT = N_PAD // NS
LANES = 16


def _sc_segment_sums(items_emb, zeros, s0, d0, w0, s1, d1, w1, s2, d2, w2):
    mesh = plsc.VectorSubcoreMesh(core_axis_name="c", subcore_axis_name="s")

    @functools.partial(
        pl.kernel,
        mesh=mesh,
        out_type=jax.ShapeDtypeStruct((3, NC, N_PAD, D), jnp.float32),
        scratch_types=(
            [pltpu.VMEM((NQ, K), jnp.int32),
             pltpu.VMEM((NQ, K), jnp.int32),
             pltpu.VMEM((NQ, K), jnp.float32)]
            + [pltpu.VMEM((K, D), jnp.float32) for _ in range(NB)]
            + [pltpu.VMEM_SHARED((N_PAD, D), jnp.float32)]
            + [pltpu.SemaphoreType.DMA for _ in range(NQ + 2 * NB + 1)]
        ),
    )
    def seg(emb_hbm, z_hbm, s0_hbm, d0_hbm, w0_hbm, s1_hbm, d1_hbm, w1_hbm,
            s2_hbm, d2_hbm, w2_hbm, out_hbm, sring, dring, wring, *rest):
        rows = list(rest[:NB])
        acc = rest[NB]
        base = NB + 1
        isem = list(rest[base:base + NQ])
        gsem = list(rest[base + NQ:base + NQ + NB])
        ssem = list(rest[base + NQ + NB:base + NQ + 2 * NB])
        zsem = rest[base + NQ + 2 * NB]

        cid = lax.axis_index("c")
        sid = lax.axis_index("s")
        wid = sid * NC + cid
        ebase = wid * EPW
        row0 = sid * RPT

        for rel, (src_hbm, dst_hbm, wr_hbm) in enumerate(
                ((s0_hbm, d0_hbm, w0_hbm), (s1_hbm, d1_hbm, w1_hbm),
                 (s2_hbm, d2_hbm, w2_hbm))):
            def load_idx(c, q):
                e0 = ebase + c * K
                pltpu.async_copy(src_hbm.at[pl.ds(e0, K)], sring.at[q],
                                 isem[q])
                pltpu.async_copy(dst_hbm.at[pl.ds(e0, K)], dring.at[q],
                                 isem[q])
                pltpu.async_copy(wr_hbm.at[pl.ds(e0, K)], wring.at[q],
                                 isem[q])

            def wait_idx(c, q):
                e0 = ebase + c * K
                pltpu.make_async_copy(src_hbm.at[pl.ds(e0, K)],
                                      sring.at[q], isem[q]).wait()
                pltpu.make_async_copy(dst_hbm.at[pl.ds(e0, K)],
                                      dring.at[q], isem[q]).wait()
                pltpu.make_async_copy(wr_hbm.at[pl.ds(e0, K)],
                                      wring.at[q], isem[q]).wait()

            pltpu.async_copy(z_hbm, acc.at[pl.ds(row0, RPT)], zsem)
            for p in range(NQ - 1):
                load_idx(p, p)
            for p in range(NB - 1):
                wait_idx(p, p)
                pltpu.async_copy(emb_hbm.at[sring.at[p]], rows[p], gsem[p])
            pltpu.make_async_copy(z_hbm, acc.at[pl.ds(row0, RPT)],
                                  zsem).wait()
            plsc.subcore_barrier()

            def outer(i, _):
                c0v = i * NQ
                for b in range(NQ):
                    c = c0v + b
                    rb = b % NB
                    rp = (b + NB - 1) % NB

                    @pl.when(c < NCHUNK)
                    def _(c=c, b=b, rb=rb, rp=rp):
                        buf = rows[rb]
                        pltpu.make_async_copy(
                            emb_hbm.at[sring.at[b]], buf, gsem[rb]).wait()

                        def scale(g, _):
                            wv = wring[b, pl.ds(g * LANES, LANES)]
                            for t in range(LANES):
                                e = g * LANES + t
                                ws = wv[t]
                                for cc in range(D // LANES):
                                    sl = pl.ds(cc * LANES, LANES)
                                    buf[e, sl] = buf[e, sl] * ws
                            return 0
                        lax.fori_loop(0, K // LANES, scale, 0)

                        @pl.when(c >= 1)
                        def _():
                            pltpu.make_async_copy(
                                rows[rp], acc.at[dring.at[(b - 1) % NQ]],
                                ssem[rp]).wait()
                        pltpu.async_copy(buf, acc.at[dring.at[b]],
                                         ssem[rb], add=True)

                        @pl.when(c + NB - 1 < NCHUNK)
                        def _():
                            q2 = (b + NB - 1) % NQ
                            wait_idx(c + NB - 1, q2)
                            pltpu.async_copy(emb_hbm.at[sring.at[q2]],
                                             rows[rp], gsem[rp])

                        @pl.when(c + NQ - 1 < NCHUNK)
                        def _():
                            q3 = (b + NQ - 1) % NQ
                            load_idx(c + NQ - 1, q3)
                return 0
            lax.fori_loop(0, NOUT, outer, 0)
            pltpu.make_async_copy(
                rows[(NCHUNK - 1) % NB],
                acc.at[dring.at[(NCHUNK - 1) % NQ]],
                ssem[(NCHUNK - 1) % NB]).wait()
            plsc.subcore_barrier()

            pltpu.sync_copy(acc.at[pl.ds(row0, RPT)],
                            out_hbm.at[rel, cid, pl.ds(row0, RPT)])
        plsc.subcore_barrier()

    return seg(items_emb, zeros, s0, d0, w0, s1, d1, w1, s2, d2, w2)


ROWS_BLK = 1000
SCALE = 1.0 / (D ** 0.5)


def _fuse_body(x_ref, parts_ref, Ii_ref, Io_ref, Ie_ref, cw_ref, cb_ref,
               out_ref):
    x = x_ref[...]
    nb = [parts_ref[r, 0] + parts_ref[r, 1] for r in range(3)]
    s = []
    for r, I_ref in enumerate((Ii_ref, Io_ref, Ie_ref)):
        h = jnp.dot(x * nb[r], I_ref[...],
                    preferred_element_type=jnp.float32,
                    precision=lax.Precision.HIGHEST)
        h = jnp.where(h > 0, h, 0.2 * h)
        s.append(jnp.sum(h, axis=1, keepdims=True) * SCALE)
    m = jnp.maximum(jnp.maximum(s[0], s[1]), s[2])
    e = [jnp.exp(sr - m) for sr in s]
    denom = e[0] + e[1] + e[2]
    neighbor = (nb[0] * e[0] + nb[1] * e[1] + nb[2] * e[2]) / denom
    out_ref[...] = x * cw_ref[0] + neighbor * cw_ref[1] + cb_ref[0]


def _tc_fuse(items_emb, parts, Ii, Io, Ie, conv_w, conv_b):
    grid = (N // ROWS_BLK,)
    return pl.pallas_call(
        _fuse_body,
        grid=grid,
        in_specs=[
            pl.BlockSpec((ROWS_BLK, D), lambda i: (i, 0)),
            pl.BlockSpec((3, NC, ROWS_BLK, D), lambda i: (0, 0, i, 0)),
            pl.BlockSpec((D, D), lambda i: (0, 0)),
            pl.BlockSpec((D, D), lambda i: (0, 0)),
            pl.BlockSpec((D, D), lambda i: (0, 0)),
            pl.BlockSpec(memory_space=pltpu.SMEM),
            pl.BlockSpec(memory_space=pltpu.SMEM),
        ],
        out_specs=pl.BlockSpec((ROWS_BLK, D), lambda i: (i, 0)),
        out_shape=jax.ShapeDtypeStruct((N, D), jnp.float32),
    )(items_emb, parts, Ii, Io, Ie, conv_w, conv_b)


def kernel(b, items_emb, p2p_in_idx, p2p_in_w, p2p_out_idx, p2p_out_w,
           e2p_in_idx, e2p_in_w, I_p2p_in, I_p2p_out, I_e2p_in, conv_w,
           conv_b):
    del b

    zeros = jnp.zeros((RPT, D), jnp.float32)
    parts = _sc_segment_sums(
        items_emb, zeros,
        p2p_in_idx[1], p2p_in_idx[0], p2p_in_w,
        p2p_out_idx[1], p2p_out_idx[0], p2p_out_w,
        e2p_in_idx[1], e2p_in_idx[0], e2p_in_w)
    return _tc_fuse(items_emb, parts, I_p2p_in, I_p2p_out, I_e2p_in,
                    conv_w, jnp.reshape(conv_b, (1,)))
